# Initial kernel scaffold; baseline (speedup 1.0000x reference)
import jax
import jax.numpy as jnp
from jax.experimental import pallas as pl


def kernel(x, edge_index, W1, b1, W2, b2):
    raise NotImplementedError("write your pallas kernel here")


# speedup vs baseline: 12.2739x; 12.2739x over previous
import functools
import jax
import jax.numpy as jnp
from jax import lax
from jax.experimental import pallas as pl
from jax.experimental.pallas import tpu as pltpu
from jax.experimental.pallas import tpu_sc as plsc

N_NODES = 10000
N_EDGES = 320000
D = 128
NC, NS = 2, 16
EPC = N_EDGES // NC
EPT = EPC // NS
CH = 80
NCHUNK = EPT // CH
BLK = 40
NBLK = N_NODES // BLK
BPT = -(-NBLK // NS)

_mesh = plsc.VectorSubcoreMesh(core_axis_name="c", subcore_axis_name="s")


def _zero_vmem_2d(ref, nrows, ncols):
    z16 = jnp.zeros((16,), jnp.float32)

    def row(r, carry):
        for j in range(ncols // 16):
            ref[r, pl.ds(j * 16, 16)] = z16
        return carry

    lax.fori_loop(0, nrows, row, 0)


def _blocks(s, fn):
    def body(j, carry):
        bi = s + j * NS

        @pl.when(bi < NBLK)
        def _():
            fn(pl.multiple_of(bi * BLK, 8))

        return carry

    lax.fori_loop(0, BPT, body, 0)


@functools.partial(
    pl.kernel,
    out_type=jax.ShapeDtypeStruct((NC, N_NODES, D), jnp.float32),
    mesh=_mesh,
    scratch_types=[
        pltpu.VMEM((CH,), jnp.int32),
        pltpu.VMEM((CH, D), jnp.float32),
        pltpu.VMEM((BLK, D), jnp.float32),
        pltpu.VMEM_SHARED((N_NODES, D), jnp.float32),
    ],
)
def _sc_degree(dst_hbm, out_hbm, idx_v, ones_v, zero_v, acc_sh):
    c = lax.axis_index("c")
    s = lax.axis_index("s")

    one16 = jnp.ones((16,), jnp.float32)

    def fill_ones(r, carry):
        for j in range(D // 16):
            ones_v[r, pl.ds(j * 16, 16)] = one16
        return carry

    lax.fori_loop(0, CH, fill_ones, 0)
    _zero_vmem_2d(zero_v, BLK, D)
    _blocks(s, lambda off: pltpu.sync_copy(zero_v, acc_sh.at[pl.ds(off, BLK)]))
    plsc.subcore_barrier()

    base = c * EPC + s * EPT

    def chunk(i, carry):
        off = pl.multiple_of(base + i * CH, 8)
        pltpu.sync_copy(dst_hbm.at[pl.ds(off, CH)], idx_v)
        pltpu.sync_copy(ones_v, acc_sh.at[idx_v], add=True)
        return carry

    lax.fori_loop(0, NCHUNK, chunk, 0)
    plsc.subcore_barrier()
    _blocks(s, lambda off: pltpu.sync_copy(
        acc_sh.at[pl.ds(off, BLK)], out_hbm.at[c, pl.ds(off, BLK)]))


@functools.partial(
    pl.kernel,
    out_type=jax.ShapeDtypeStruct((NC, N_NODES, D), jnp.float32),
    mesh=_mesh,
    scratch_types=[
        pltpu.VMEM((CH,), jnp.int32),
        pltpu.VMEM((CH,), jnp.int32),
        pltpu.VMEM((CH, D), jnp.float32),
        pltpu.VMEM((BLK, D), jnp.float32),
        pltpu.VMEM_SHARED((N_NODES, D), jnp.float32),
        pltpu.SemaphoreType.DMA,
    ],
)
def _sc_scatter(g_hbm, src_hbm, dst_hbm, out_hbm,
                isrc_v, idst_v, rows_v, zero_v, acc_sh, sem):
    c = lax.axis_index("c")
    s = lax.axis_index("s")

    _zero_vmem_2d(zero_v, BLK, D)
    _blocks(s, lambda off: pltpu.sync_copy(zero_v, acc_sh.at[pl.ds(off, BLK)]))
    plsc.subcore_barrier()

    base = c * EPC + s * EPT

    def chunk(i, carry):
        off = pl.multiple_of(base + i * CH, 8)
        pltpu.sync_copy(src_hbm.at[pl.ds(off, CH)], isrc_v)
        pltpu.async_copy(g_hbm.at[isrc_v], rows_v, sem).wait()
        pltpu.sync_copy(dst_hbm.at[pl.ds(off, CH)], idst_v)
        pltpu.sync_copy(rows_v, acc_sh.at[idst_v], add=True)
        return carry

    lax.fori_loop(0, NCHUNK, chunk, 0)
    plsc.subcore_barrier()
    _blocks(s, lambda off: pltpu.sync_copy(
        acc_sh.at[pl.ds(off, BLK)], out_hbm.at[c, pl.ds(off, BLK)]))


_BR = 1000
_GRID = N_NODES // _BR


def _dis_of(degp):
    return lax.rsqrt(degp[0] + degp[1] + 1.0)


def _tc1_body(x_ref, w_ref, degp_ref, o_ref):
    dis = _dis_of(degp_ref[...])
    h = jnp.dot(x_ref[...], w_ref[...], preferred_element_type=jnp.float32)
    o_ref[...] = h * dis


def _tc2_body(s_ref, g_ref, degp_ref, b_ref, w_ref, o_ref):
    dis = _dis_of(degp_ref[...])
    tot = s_ref[0] + s_ref[1] + g_ref[...]
    z = jnp.maximum(tot * dis + b_ref[...], 0.0)
    h = jnp.dot(z, w_ref[...], preferred_element_type=jnp.float32)
    o_ref[...] = h * dis


def _tc3_body(s_ref, g_ref, degp_ref, b_ref, o_ref):
    dis = _dis_of(degp_ref[...])
    tot = s_ref[0] + s_ref[1] + g_ref[...]
    o_ref[...] = jnp.maximum(tot * dis + b_ref[...], 0.0)


_spec_rows = pl.BlockSpec((_BR, D), lambda i: (i, 0))
_spec_S = pl.BlockSpec((NC, _BR, D), lambda i: (0, i, 0))
_spec_degp = pl.BlockSpec((NC, _BR, D), lambda i: (0, i, 0))
_spec_w = pl.BlockSpec((D, D), lambda i: (0, 0))
_spec_b = pl.BlockSpec((1, D), lambda i: (0, 0))

_tc1 = pl.pallas_call(
    _tc1_body, grid=(_GRID,),
    in_specs=[_spec_rows, _spec_w, _spec_degp],
    out_specs=_spec_rows,
    out_shape=jax.ShapeDtypeStruct((N_NODES, D), jnp.float32),
)
_tc2 = pl.pallas_call(
    _tc2_body, grid=(_GRID,),
    in_specs=[_spec_S, _spec_rows, _spec_degp, _spec_b, _spec_w],
    out_specs=_spec_rows,
    out_shape=jax.ShapeDtypeStruct((N_NODES, D), jnp.float32),
)
_tc3 = pl.pallas_call(
    _tc3_body, grid=(_GRID,),
    in_specs=[_spec_S, _spec_rows, _spec_degp, _spec_b],
    out_specs=_spec_rows,
    out_shape=jax.ShapeDtypeStruct((N_NODES, D), jnp.float32),
)


@jax.jit
def kernel(x, edge_index, W1, b1, W2, b2):
    src = edge_index[0].astype(jnp.int32)
    dst = edge_index[1].astype(jnp.int32)
    b1r = b1.reshape(1, D)
    b2r = b2.reshape(1, D)

    degp = _sc_degree(dst)
    g1 = _tc1(x, W1, degp)
    S1 = _sc_scatter(g1, src, dst)
    g2 = _tc2(S1, g1, degp, b1r, W2)
    S2 = _sc_scatter(g2, src, dst)
    return _tc3(S2, g2, degp, b2r)
